# baseline (speedup 1.0000x reference)
import jax
import jax.numpy as jnp
from jax.experimental import pallas as pl
from jax.experimental.pallas import tpu as pltpu

K = 20
P = 20


def _conv(x, W):
    return jnp.einsum('oc,bcn->bon', W, x)


def _gn(x, gamma, beta, G, eps=1e-5):
    B, C, N = x.shape
    xg = x.reshape(B, G, C // G, N)
    m = jnp.mean(xg, axis=(2, 3), keepdims=True)
    v = jnp.var(xg, axis=(2, 3), keepdims=True)
    xg = (xg - m) / jnp.sqrt(v + eps)
    x = xg.reshape(B, C, N)
    return x * gamma[None, :, None] + beta[None, :, None]


def _lrelu(x):
    return jnp.where(x >= 0, x, 0.2 * x)


def _block(x, W, g, b, G):
    return _lrelu(_gn(_conv(x, W), g, b, G))


def _knn(x, k):
    inner = -2.0 * jnp.einsum('bcn,bcm->bnm', x, x)
    xx = jnp.sum(x ** 2, axis=1, keepdims=True)
    pd = -xx - inner - jnp.transpose(xx, (0, 2, 1))
    _, idx = jax.lax.top_k(pd, k)
    return idx


def _gather(f, idx):
    return jax.vmap(lambda fb, ib: fb[:, ib])(f, idx)


def _point_conv(x_in, Wa, ga, ba, Wb, gb, bb, G, idx):
    fa = _block(x_in, Wa, ga, ba, G)
    fb = _block(x_in, Wb, gb, bb, G)
    nf = _gather(fa, idx)
    agg = jnp.max(nf, axis=-1)
    return agg + fb


def _final_block_kernel(x_ref, w_ref, g_ref, b_ref, o_ref):
    x = x_ref[0]
    w = w_ref[...]
    y = jnp.dot(w, x, preferred_element_type=jnp.float32)
    O, N = y.shape
    G = 16
    yg = y.reshape(G, O // G, N)
    m = jnp.mean(yg, axis=(1, 2), keepdims=True)
    v = jnp.mean((yg - m) ** 2, axis=(1, 2), keepdims=True)
    yg = (yg - m) / jnp.sqrt(v + 1e-5)
    y = yg.reshape(O, N)
    y = y * g_ref[...][:, None] + b_ref[...][:, None]
    o_ref[0] = jnp.where(y >= 0, y, 0.2 * y)


def _final_block(x, W, g, b):
    B, C, N = x.shape
    O = W.shape[0]
    return pl.pallas_call(
        _final_block_kernel,
        grid=(B,),
        in_specs=[
            pl.BlockSpec((1, C, N), lambda i: (i, 0, 0)),
            pl.BlockSpec((O, C), lambda i: (0, 0)),
            pl.BlockSpec((O,), lambda i: (0,)),
            pl.BlockSpec((O,), lambda i: (0,)),
        ],
        out_specs=pl.BlockSpec((1, O, N), lambda i: (i, 0, 0)),
        out_shape=jax.ShapeDtypeStruct((B, O, N), jnp.float32),
    )(x, W, g, b)


def kernel(x, W1a, g1a, b1a, W1b, g1b, b1b, W2a, g2a, b2a, W2b, g2b, b2b,
           W3a, g3a, b3a, W3b, g3b, b3b, W4a, g4a, b4a, W4b, g4b, b4b,
           W5a, g5a, b5a, W5b, g5b, b5b):
    xt = jnp.transpose(x, (0, 2, 1))
    pool_size = K + 3 * P
    idx_pool = _knn(xt, pool_size)
    idx1 = idx_pool[:, :, :K]
    idx2 = idx_pool[:, :, :K + P]
    idx3 = idx_pool[:, :, :K + 2 * P]
    idx4 = idx_pool
    x1 = _point_conv(xt, W1a, g1a, b1a, W1b, g1b, b1b, 8, idx1)
    x2 = _point_conv(x1, W2a, g2a, b2a, W2b, g2b, b2b, 8, idx2)
    x3 = _point_conv(x2, W3a, g3a, b3a, W3b, g3b, b3b, 8, idx3)
    x4 = _point_conv(x3, W4a, g4a, b4a, W4b, g4b, b4b, 16, idx4)
    xc = jnp.concatenate((x1, x2, x3, x4), axis=1)
    x5 = _block(xc, W5a, g5a, b5a, 16)
    x6 = _final_block(x5, W5b, g5b, b5b)
    return jnp.transpose(x6, (0, 2, 1))


def _squeeze_blockspec_fix():
    return None


# speedup vs baseline: 22.8801x; 22.8801x over previous
import functools

import jax
import jax.numpy as jnp
from jax import lax
from jax.experimental import pallas as pl
from jax.experimental.pallas import tpu as pltpu
from jax.experimental.pallas import tpu_sc as plsc

K = 20
P = 20
B = 8
N = 2048
NROWS = B * N
NC = 2
NS = 16
NW = NC * NS
ROWS_PER_W = NROWS // NW


def _pd_kernel(x8_ref, xt8_ref, norms_ref, pd_ref, cm_ref):
    xb = x8_ref[0]
    xbt = xt8_ref[0]
    g = jnp.dot(xb, xbt, preferred_element_type=jnp.float32)
    pd = 2.0 * g - norms_ref[0]
    pd_ref[0] = pd
    cm = pd[:, 0:256]
    for c in range(1, 8):
        cm = jnp.maximum(cm, pd[:, c * 256:(c + 1) * 256])
    cm_ref[0] = cm


def _pd_chunkmax(x8, xt8, norms):
    return pl.pallas_call(
        _pd_kernel,
        grid=(B,),
        in_specs=[
            pl.BlockSpec((1, N, 8), lambda i: (i, 0, 0)),
            pl.BlockSpec((1, 8, N), lambda i: (i, 0, 0)),
            pl.BlockSpec((1, 1, N), lambda i: (i, 0, 0)),
        ],
        out_specs=[
            pl.BlockSpec((1, N, N), lambda i: (i, 0, 0)),
            pl.BlockSpec((1, N, 256), lambda i: (i, 0, 0)),
        ],
        out_shape=[
            jax.ShapeDtypeStruct((B, N, N), jnp.float32),
            jax.ShapeDtypeStruct((B, N, 256), jnp.float32),
        ],
    )(x8, xt8, norms)


def _gn_lrelu(y, G, g_ref, b_ref):
    O = y.shape[0]
    yg = y.reshape(G, O // G, N)
    m = jnp.mean(yg, axis=(1, 2), keepdims=True)
    v = jnp.mean((yg - m) ** 2, axis=(1, 2), keepdims=True)
    yg = (yg - m) * lax.rsqrt(v + 1e-5)
    y = yg.reshape(O, N)
    y = y * g_ref[...][:, None] + b_ref[...][:, None]
    return jnp.where(y >= 0, y, 0.2 * y)


def _block_pair(x_s, wa_ref, ga_ref, ba_ref, wb_ref, gb_ref, bb_ref, G):
    fa = _gn_lrelu(jnp.dot(wa_ref[...], x_s, preferred_element_type=jnp.float32),
                   G, ga_ref, ba_ref)
    fb = _gn_lrelu(jnp.dot(wb_ref[...], x_s, preferred_element_type=jnp.float32),
                   G, gb_ref, bb_ref)
    return fa, fb


def _conv1_kernel(xt8_ref, wa_ref, ga_ref, ba_ref, wb_ref, gb_ref, bb_ref,
                  faT_ref, fb_ref):
    x_s = xt8_ref[0]
    fa, fb = _block_pair(x_s, wa_ref, ga_ref, ba_ref, wb_ref, gb_ref, bb_ref, 8)
    faT = fa.T
    O = faT.shape[1]
    CP = faT_ref.shape[2]
    if CP > O:
        faT = jnp.concatenate(
            [faT, jnp.zeros((faT.shape[0], CP - O), jnp.float32)], axis=1)
    faT_ref[0] = faT
    fb_ref[0] = fb


def _conv1(xt8, Wa8, ga, ba, Wb8, gb, bb, CP):
    O = Wa8.shape[0]
    return pl.pallas_call(
        _conv1_kernel,
        grid=(B,),
        in_specs=[
            pl.BlockSpec((1, 8, N), lambda i: (i, 0, 0)),
            pl.BlockSpec(Wa8.shape, lambda i: (0, 0)),
            pl.BlockSpec((O,), lambda i: (0,)),
            pl.BlockSpec((O,), lambda i: (0,)),
            pl.BlockSpec(Wb8.shape, lambda i: (0, 0)),
            pl.BlockSpec((O,), lambda i: (0,)),
            pl.BlockSpec((O,), lambda i: (0,)),
        ],
        out_specs=[
            pl.BlockSpec((1, N, CP), lambda i: (i, 0, 0)),
            pl.BlockSpec((1, O, N), lambda i: (i, 0, 0)),
        ],
        out_shape=[
            jax.ShapeDtypeStruct((B, N, CP), jnp.float32),
            jax.ShapeDtypeStruct((B, O, N), jnp.float32),
        ],
    )(xt8, Wa8, ga, ba, Wb8, gb, bb)


def _conv_stage_kernel(G, agg_ref, fbp_ref, wa_ref, ga_ref, ba_ref,
                       wb_ref, gb_ref, bb_ref, faT_ref, fb_ref, xprev_ref):
    x_s = fbp_ref[0] + agg_ref[0].T
    xprev_ref[0] = x_s
    fa, fb = _block_pair(x_s, wa_ref, ga_ref, ba_ref, wb_ref, gb_ref, bb_ref, G)
    faT = fa.T
    O = faT.shape[1]
    CP = faT_ref.shape[2]
    if CP > O:
        faT = jnp.concatenate(
            [faT, jnp.zeros((faT.shape[0], CP - O), jnp.float32)], axis=1)
    faT_ref[0] = faT
    fb_ref[0] = fb


def _conv_stage(agg, fbp, Wa, ga, ba, Wb, gb, bb, G, CP):
    C = Wa.shape[1]
    CA = agg.shape[2]
    O = Wa.shape[0]
    return pl.pallas_call(
        functools.partial(_conv_stage_kernel, G),
        grid=(B,),
        in_specs=[
            pl.BlockSpec((1, N, CA), lambda i: (i, 0, 0)),
            pl.BlockSpec((1, C, N), lambda i: (i, 0, 0)),
            pl.BlockSpec((O, C), lambda i: (0, 0)),
            pl.BlockSpec((O,), lambda i: (0,)),
            pl.BlockSpec((O,), lambda i: (0,)),
            pl.BlockSpec((O, C), lambda i: (0, 0)),
            pl.BlockSpec((O,), lambda i: (0,)),
            pl.BlockSpec((O,), lambda i: (0,)),
        ],
        out_specs=[
            pl.BlockSpec((1, N, CP), lambda i: (i, 0, 0)),
            pl.BlockSpec((1, O, N), lambda i: (i, 0, 0)),
            pl.BlockSpec((1, C, N), lambda i: (i, 0, 0)),
        ],
        out_shape=[
            jax.ShapeDtypeStruct((B, N, CP), jnp.float32),
            jax.ShapeDtypeStruct((B, O, N), jnp.float32),
            jax.ShapeDtypeStruct((B, C, N), jnp.float32),
        ],
    )(agg, fbp, Wa, ga, ba, Wb, gb, bb)


def _head_kernel(agg4_ref, fb4_ref, x1_ref, x2_ref, x3_ref,
                 w5a_ref, g5a_ref, b5a_ref, w5b_ref, g5b_ref, b5b_ref, out_ref):
    x4 = fb4_ref[0] + agg4_ref[0].T
    w = w5a_ref[...]
    y = (jnp.dot(w[:, 0:64], x1_ref[0], preferred_element_type=jnp.float32)
         + jnp.dot(w[:, 64:128], x2_ref[0], preferred_element_type=jnp.float32)
         + jnp.dot(w[:, 128:256], x3_ref[0], preferred_element_type=jnp.float32)
         + jnp.dot(w[:, 256:512], x4, preferred_element_type=jnp.float32))
    x5 = _gn_lrelu(y, 16, g5a_ref, b5a_ref)
    y6 = jnp.dot(w5b_ref[...], x5, preferred_element_type=jnp.float32)
    x6 = _gn_lrelu(y6, 16, g5b_ref, b5b_ref)
    out_ref[0] = x6.T


def _head(agg4, fb4, x1, x2, x3, W5a, g5a, b5a, W5b, g5b, b5b):
    return pl.pallas_call(
        _head_kernel,
        grid=(B,),
        in_specs=[
            pl.BlockSpec((1, N, 256), lambda i: (i, 0, 0)),
            pl.BlockSpec((1, 256, N), lambda i: (i, 0, 0)),
            pl.BlockSpec((1, 64, N), lambda i: (i, 0, 0)),
            pl.BlockSpec((1, 64, N), lambda i: (i, 0, 0)),
            pl.BlockSpec((1, 128, N), lambda i: (i, 0, 0)),
            pl.BlockSpec((1024, 512), lambda i: (0, 0)),
            pl.BlockSpec((1024,), lambda i: (0,)),
            pl.BlockSpec((1024,), lambda i: (0,)),
            pl.BlockSpec((512, 1024), lambda i: (0, 0)),
            pl.BlockSpec((512,), lambda i: (0,)),
            pl.BlockSpec((512,), lambda i: (0,)),
        ],
        out_specs=pl.BlockSpec((1, N, 512), lambda i: (i, 0, 0)),
        out_shape=jax.ShapeDtypeStruct((B, N, 512), jnp.float32),
    )(agg4, fb4, x1, x2, x3, W5a, g5a, b5a, W5b, g5b, b5b)


def _make_agg(k, C, CP):
    CH = C // 16
    OUT_CH = 64

    mesh = plsc.VectorSubcoreMesh(core_axis_name="c", subcore_axis_name="s",
                                  num_cores=NC, num_subcores=NS)

    def point_max(rows_buf):
        accs = [rows_buf[0, pl.ds(c * 16, 16)] for c in range(CH)]

        def body(r4, accs):
            accs = list(accs)
            r = 1 + r4 * 4
            for u in range(4):
                for c in range(CH):
                    accs[c] = jnp.maximum(
                        accs[c], rows_buf[r + u, pl.ds(c * 16, 16)])
            return tuple(accs)

        n4 = (k - 1) // 4
        accs = list(lax.fori_loop(0, n4, body, tuple(accs), unroll=False))
        for r in range(1 + n4 * 4, k):
            for c in range(CH):
                accs[c] = jnp.maximum(accs[c], rows_buf[r, pl.ds(c * 16, 16)])
        return accs

    @functools.partial(
        pl.kernel,
        out_type=jax.ShapeDtypeStruct((NROWS, C), jnp.float32),
        mesh=mesh,
        scratch_types=[
            pltpu.VMEM((ROWS_PER_W, 80), jnp.int32),
            pltpu.VMEM((k, CP), jnp.float32),
            pltpu.VMEM((k, CP), jnp.float32),
            pltpu.VMEM((OUT_CH, C), jnp.float32),
            pltpu.SemaphoreType.DMA,
            pltpu.SemaphoreType.DMA,
        ],
    )
    def agg(fa_hbm, idx_hbm, out_hbm, idx_v, buf0, buf1, out_v, sem0, sem1):
        wid = lax.axis_index("s") * NC + lax.axis_index("c")
        base = wid * ROWS_PER_W
        pltpu.sync_copy(idx_hbm.at[pl.ds(base, ROWS_PER_W), :], idx_v)

        def issue(p, buf, sem):
            pc = jnp.minimum(p, ROWS_PER_W - 1)
            pltpu.async_copy(fa_hbm.at[idx_v.at[pc, pl.ds(0, k)]], buf, sem)

        def waitbuf(buf, sem):
            pltpu.make_async_copy(fa_hbm.at[idx_v.at[0, pl.ds(0, k)]], buf,
                                  sem).wait()

        issue(0, buf0, sem0)
        issue(1, buf1, sem1)

        def flush_chunk(o, _):
            def pair(t, _):
                p0 = o * OUT_CH + 2 * t
                waitbuf(buf0, sem0)
                accs = point_max(buf0)
                for c in range(CH):
                    out_v[2 * t, pl.ds(c * 16, 16)] = accs[c]
                issue(p0 + 2, buf0, sem0)
                waitbuf(buf1, sem1)
                accs = point_max(buf1)
                for c in range(CH):
                    out_v[2 * t + 1, pl.ds(c * 16, 16)] = accs[c]
                issue(p0 + 3, buf1, sem1)
                return 0

            lax.fori_loop(0, OUT_CH // 2, pair, 0, unroll=False)
            pltpu.sync_copy(
                out_v, out_hbm.at[pl.ds(base + o * OUT_CH, OUT_CH), :])
            return 0

        lax.fori_loop(0, ROWS_PER_W // OUT_CH, flush_chunk, 0, unroll=False)
        waitbuf(buf0, sem0)
        waitbuf(buf1, sem1)

    return agg


def _sc_agg(faT, idxg, k, C):
    CP = faT.shape[2]
    return _make_agg(k, C, CP)(faT.reshape(NROWS, CP), idxg)


_NEG = -3.0e38


def _ce(mk, mp, i, j):
    m = mk[i] >= mk[j]
    hik = jnp.where(m, mk[i], mk[j])
    lok = jnp.where(m, mk[j], mk[i])
    hip = jnp.where(m, mp[i], mp[j])
    lop = jnp.where(m, mp[j], mp[i])
    return hik, lok, hip, lop


def _merge16(Lk, Lp, vk, vp):
    neg = jnp.full((16,), _NEG, jnp.float32)
    zero = jnp.zeros((16,), jnp.int32)
    mk = [Lk[0], Lk[1], Lk[2], Lk[3], Lk[4], neg, neg, vk]
    mp = [Lp[0], Lp[1], Lp[2], Lp[3], Lp[4], zero, zero, vp]
    for (i, j) in ((0, 4), (1, 5), (2, 6), (3, 7)):
        mk[i], mk[j], mp[i], mp[j] = _ce(mk, mp, i, j)
    for (i, j) in ((0, 2), (1, 3), (4, 6), (5, 7)):
        mk[i], mk[j], mp[i], mp[j] = _ce(mk, mp, i, j)
    for (i, j) in ((0, 1), (2, 3), (4, 5)):
        mk[i], mk[j], mp[i], mp[j] = _ce(mk, mp, i, j)
    outk, outp = [], []
    for i in range(5):
        ks, ps = plsc.sort_key_val(mk[i], mp[i], descending=True)
        outk.append(ks)
        outp.append(ps)
    return outk, outp


def _make_topk():
    OUT_CH = 64
    CAP = N + 16

    mesh = plsc.VectorSubcoreMesh(core_axis_name="c", subcore_axis_name="s",
                                  num_cores=NC, num_subcores=NS)

    @functools.partial(
        pl.kernel,
        out_type=jax.ShapeDtypeStruct((NROWS, 80), jnp.int32),
        mesh=mesh,
        scratch_types=[
            pltpu.VMEM((N,), jnp.float32),
            pltpu.VMEM((N,), jnp.float32),
            pltpu.VMEM((256,), jnp.float32),
            pltpu.VMEM((256,), jnp.float32),
            pltpu.VMEM((CAP,), jnp.float32),
            pltpu.VMEM((CAP,), jnp.int32),
            pltpu.VMEM((OUT_CH, 80), jnp.int32),
            pltpu.SemaphoreType.DMA,
            pltpu.SemaphoreType.DMA,
            pltpu.SemaphoreType.DMA,
            pltpu.SemaphoreType.DMA,
        ],
        compiler_params=pltpu.CompilerParams(needs_layout_passes=False),
    )
    def topk(pd_hbm, cm_hbm, out_hbm, row0, row1, cmb0, cmb1,
             cand_v, cand_i, out_v, semr0, semr1, semc0, semc1):
        wid = lax.axis_index("s") * NC + lax.axis_index("c")
        base = wid * ROWS_PER_W
        joff = (base // N) * N

        iota = lax.broadcasted_iota(jnp.int32, (16,), 0)

        def issue(p, rowb, cmb, semr, semc):
            pc = jnp.minimum(p, ROWS_PER_W - 1)
            pltpu.async_copy(pd_hbm.at[base + pc, :], rowb, semr)
            pltpu.async_copy(cm_hbm.at[base + pc, :], cmb, semc)

        def waitb(rowb, cmb, semr, semc):
            pltpu.make_async_copy(pd_hbm.at[base, :], rowb, semr).wait()
            pltpu.make_async_copy(cm_hbm.at[base, :], cmb, semc).wait()

        def process(p, rowb, cmb):
            neg = jnp.full((16,), _NEG, jnp.float32)
            r = [neg, neg, neg, neg, neg]
            for i in range(16):
                v = cmb[pl.ds(i * 16, 16)]
                for s in range(5):
                    hi = jnp.maximum(r[s], v)
                    v = jnp.minimum(r[s], v)
                    r[s] = hi
            t = jnp.min(r[4])
            tv = jnp.full((16,), t, jnp.float32)

            def compact(jb, off):
                for u in range(8):
                    j0 = (jb * 8 + u) * 16
                    v = rowb[pl.ds(j0, 16)]
                    m = v >= tv
                    plsc.store_compressed(cand_v.at[pl.ds(off, 16)], v, mask=m)
                    plsc.store_compressed(cand_i.at[pl.ds(off, 16)],
                                          iota + j0, mask=m)
                    off = off + jnp.sum(m.astype(jnp.int32))
                return off

            off = lax.fori_loop(0, 16, compact, jnp.int32(0), unroll=False)
            cand_v[pl.ds(off, 16)] = neg

            zero = jnp.zeros((16,), jnp.int32)
            init = (neg, neg, neg, neg, neg, zero, zero, zero, zero, zero)

            def sel(i, carry):
                Lk = list(carry[0:5])
                Lp = list(carry[5:10])
                vk = cand_v[pl.ds(i * 16, 16)]
                vp = cand_i[pl.ds(i * 16, 16)]
                lmin = jnp.min(Lk[4])

                def do_merge(_):
                    vs, ps = plsc.sort_key_val(vk, vp, descending=False)
                    nk, np_ = _merge16(Lk, Lp, vs, ps)
                    return tuple(nk) + tuple(np_)

                def skip(_):
                    return tuple(Lk) + tuple(Lp)

                cnt = jnp.sum((vk >= jnp.full((16,), lmin)).astype(jnp.int32))
                return lax.cond(cnt > 0, do_merge, skip, 0)

            nv = (off + 15) // 16
            fin = lax.fori_loop(0, nv, sel, init, unroll=False)

            prow = p % OUT_CH
            for g in range(5):
                out_v[prow, pl.ds(g * 16, 16)] = fin[5 + g] + joff

        issue(0, row0, cmb0, semr0, semc0)
        issue(1, row1, cmb1, semr1, semc1)

        def flush_chunk(o, _):
            def pair(tt, _):
                p0 = o * OUT_CH + 2 * tt
                waitb(row0, cmb0, semr0, semc0)
                process(p0, row0, cmb0)
                issue(p0 + 2, row0, cmb0, semr0, semc0)
                waitb(row1, cmb1, semr1, semc1)
                process(p0 + 1, row1, cmb1)
                issue(p0 + 3, row1, cmb1, semr1, semc1)
                return 0

            lax.fori_loop(0, OUT_CH // 2, pair, 0, unroll=False)
            pltpu.sync_copy(
                out_v, out_hbm.at[pl.ds(base + o * OUT_CH, OUT_CH), :])
            return 0

        lax.fori_loop(0, ROWS_PER_W // OUT_CH, flush_chunk, 0, unroll=False)
        waitb(row0, cmb0, semr0, semc0)
        waitb(row1, cmb1, semr1, semc1)

    return topk


def _sc_topk(pd, cm):
    return _make_topk()(pd.reshape(NROWS, N), cm.reshape(NROWS, 256))


def kernel(x, W1a, g1a, b1a, W1b, g1b, b1b, W2a, g2a, b2a, W2b, g2b, b2b,
           W3a, g3a, b3a, W3b, g3b, b3b, W4a, g4a, b4a, W4b, g4b, b4b,
           W5a, g5a, b5a, W5b, g5b, b5b):
    xt = jnp.transpose(x, (0, 2, 1))
    x8 = jnp.pad(x, ((0, 0), (0, 0), (0, 5)))
    xt8 = jnp.pad(xt, ((0, 0), (0, 5), (0, 0)))
    norms = jnp.sum(x * x, axis=-1)[:, None, :]
    Wa8 = jnp.pad(W1a, ((0, 0), (0, 5)))
    Wb8 = jnp.pad(W1b, ((0, 0), (0, 5)))

    pd, cm = _pd_chunkmax(x8, xt8, norms)
    idxg = _sc_topk(pd, cm)

    fa1T, fb1 = _conv1(xt8, Wa8, g1a, b1a, Wb8, g1b, b1b, 128)
    agg1 = _sc_agg(fa1T, idxg, K, 64).reshape(B, N, 64)
    fa2T, fb2, x1 = _conv_stage(agg1, fb1, W2a, g2a, b2a, W2b, g2b, b2b, 8, 128)
    agg2 = _sc_agg(fa2T, idxg, K + P, 64).reshape(B, N, 64)
    fa3T, fb3, x2 = _conv_stage(agg2, fb2, W3a, g3a, b3a, W3b, g3b, b3b, 8, 128)
    agg3 = _sc_agg(fa3T, idxg, K + 2 * P, 128).reshape(B, N, 128)
    fa4T, fb4, x3 = _conv_stage(agg3, fb3, W4a, g4a, b4a, W4b, g4b, b4b, 16, 256)
    agg4 = _sc_agg(fa4T, idxg, K + 3 * P, 256).reshape(B, N, 256)

    return _head(agg4, fb4, x1, x2, x3, W5a, g5a, b5a, W5b, g5b, b5b)


# speedup vs baseline: 25.4166x; 1.1109x over previous
import functools

import jax
import jax.numpy as jnp
from jax import lax
from jax.experimental import pallas as pl
from jax.experimental.pallas import tpu as pltpu
from jax.experimental.pallas import tpu_sc as plsc

K = 20
P = 20
B = 8
N = 2048
NROWS = B * N
NC = 2
NS = 16
NW = NC * NS
ROWS_PER_W = NROWS // NW


def _pd_kernel(x8_ref, xt8_ref, norms_ref, pd_ref, cm_ref):
    xb = x8_ref[0]
    xbt = xt8_ref[0]
    g = jnp.dot(xb, xbt, preferred_element_type=jnp.float32)
    pd = 2.0 * g - norms_ref[0]
    pd_ref[0] = pd
    cm = pd[:, 0:256]
    for c in range(1, 8):
        cm = jnp.maximum(cm, pd[:, c * 256:(c + 1) * 256])
    cm_ref[0] = cm


def _pd_chunkmax(x8, xt8, norms):
    return pl.pallas_call(
        _pd_kernel,
        grid=(B,),
        in_specs=[
            pl.BlockSpec((1, N, 8), lambda i: (i, 0, 0)),
            pl.BlockSpec((1, 8, N), lambda i: (i, 0, 0)),
            pl.BlockSpec((1, 1, N), lambda i: (i, 0, 0)),
        ],
        out_specs=[
            pl.BlockSpec((1, N, N), lambda i: (i, 0, 0)),
            pl.BlockSpec((1, N, 256), lambda i: (i, 0, 0)),
        ],
        out_shape=[
            jax.ShapeDtypeStruct((B, N, N), jnp.float32),
            jax.ShapeDtypeStruct((B, N, 256), jnp.float32),
        ],
    )(x8, xt8, norms)


def _gn_lrelu(y, G, g_ref, b_ref):
    O = y.shape[0]
    yg = y.reshape(G, O // G, N)
    m = jnp.mean(yg, axis=(1, 2), keepdims=True)
    v = jnp.mean((yg - m) ** 2, axis=(1, 2), keepdims=True)
    yg = (yg - m) * lax.rsqrt(v + 1e-5)
    y = yg.reshape(O, N)
    y = y * g_ref[...][:, None] + b_ref[...][:, None]
    return jnp.where(y >= 0, y, 0.2 * y)


def _block_pair(x_s, wa_ref, ga_ref, ba_ref, wb_ref, gb_ref, bb_ref, G):
    fa = _gn_lrelu(jnp.dot(wa_ref[...], x_s, preferred_element_type=jnp.float32),
                   G, ga_ref, ba_ref)
    fb = _gn_lrelu(jnp.dot(wb_ref[...], x_s, preferred_element_type=jnp.float32),
                   G, gb_ref, bb_ref)
    return fa, fb


def _conv1_kernel(xt8_ref, wa_ref, ga_ref, ba_ref, wb_ref, gb_ref, bb_ref,
                  faT_ref, fb_ref):
    x_s = xt8_ref[0]
    fa, fb = _block_pair(x_s, wa_ref, ga_ref, ba_ref, wb_ref, gb_ref, bb_ref, 8)
    faT = fa.T
    O = faT.shape[1]
    CP = faT_ref.shape[2]
    if CP > O:
        faT = jnp.concatenate(
            [faT, jnp.zeros((faT.shape[0], CP - O), jnp.float32)], axis=1)
    faT_ref[0] = faT
    fb_ref[0] = fb


def _conv1(xt8, Wa8, ga, ba, Wb8, gb, bb, CP):
    O = Wa8.shape[0]
    return pl.pallas_call(
        _conv1_kernel,
        grid=(B,),
        in_specs=[
            pl.BlockSpec((1, 8, N), lambda i: (i, 0, 0)),
            pl.BlockSpec(Wa8.shape, lambda i: (0, 0)),
            pl.BlockSpec((O,), lambda i: (0,)),
            pl.BlockSpec((O,), lambda i: (0,)),
            pl.BlockSpec(Wb8.shape, lambda i: (0, 0)),
            pl.BlockSpec((O,), lambda i: (0,)),
            pl.BlockSpec((O,), lambda i: (0,)),
        ],
        out_specs=[
            pl.BlockSpec((1, N, CP), lambda i: (i, 0, 0)),
            pl.BlockSpec((1, O, N), lambda i: (i, 0, 0)),
        ],
        out_shape=[
            jax.ShapeDtypeStruct((B, N, CP), jnp.float32),
            jax.ShapeDtypeStruct((B, O, N), jnp.float32),
        ],
    )(xt8, Wa8, ga, ba, Wb8, gb, bb)


def _conv_stage_kernel(G, agg_ref, fbp_ref, wa_ref, ga_ref, ba_ref,
                       wb_ref, gb_ref, bb_ref, faT_ref, fb_ref, xprev_ref):
    x_s = fbp_ref[0] + agg_ref[0].T
    xprev_ref[0] = x_s
    fa, fb = _block_pair(x_s, wa_ref, ga_ref, ba_ref, wb_ref, gb_ref, bb_ref, G)
    faT = fa.T
    O = faT.shape[1]
    CP = faT_ref.shape[2]
    if CP > O:
        faT = jnp.concatenate(
            [faT, jnp.zeros((faT.shape[0], CP - O), jnp.float32)], axis=1)
    faT_ref[0] = faT
    fb_ref[0] = fb


def _conv_stage(agg, fbp, Wa, ga, ba, Wb, gb, bb, G, CP):
    C = Wa.shape[1]
    CA = agg.shape[2]
    O = Wa.shape[0]
    return pl.pallas_call(
        functools.partial(_conv_stage_kernel, G),
        grid=(B,),
        in_specs=[
            pl.BlockSpec((1, N, CA), lambda i: (i, 0, 0)),
            pl.BlockSpec((1, C, N), lambda i: (i, 0, 0)),
            pl.BlockSpec((O, C), lambda i: (0, 0)),
            pl.BlockSpec((O,), lambda i: (0,)),
            pl.BlockSpec((O,), lambda i: (0,)),
            pl.BlockSpec((O, C), lambda i: (0, 0)),
            pl.BlockSpec((O,), lambda i: (0,)),
            pl.BlockSpec((O,), lambda i: (0,)),
        ],
        out_specs=[
            pl.BlockSpec((1, N, CP), lambda i: (i, 0, 0)),
            pl.BlockSpec((1, O, N), lambda i: (i, 0, 0)),
            pl.BlockSpec((1, C, N), lambda i: (i, 0, 0)),
        ],
        out_shape=[
            jax.ShapeDtypeStruct((B, N, CP), jnp.float32),
            jax.ShapeDtypeStruct((B, O, N), jnp.float32),
            jax.ShapeDtypeStruct((B, C, N), jnp.float32),
        ],
    )(agg, fbp, Wa, ga, ba, Wb, gb, bb)


def _conv4_kernel(G, agg_ref, fbp_ref, wa_ref, ga_ref, ba_ref,
                  wb_ref, gb_ref, bb_ref, faP_ref, fb_ref, xprev_ref):
    x_s = fbp_ref[0] + agg_ref[0].T
    xprev_ref[0] = x_s
    fa, fb = _block_pair(x_s, wa_ref, ga_ref, ba_ref, wb_ref, gb_ref, bb_ref, G)
    u = lax.bitcast_convert_type(fa, jnp.uint32)
    bf = (u + jnp.uint32(0x7FFF) + ((u >> 16) & jnp.uint32(1))) >> 16
    O = fa.shape[0]
    bfp = bf.reshape(O // 2, 2, N)
    packed = bfp[:, 0, :] | (bfp[:, 1, :] << 16)
    faP_ref[0] = lax.bitcast_convert_type(packed.T, jnp.int32)
    fb_ref[0] = fb


def _conv_stage4(agg, fbp, Wa, ga, ba, Wb, gb, bb, G):
    C = Wa.shape[1]
    CA = agg.shape[2]
    O = Wa.shape[0]
    return pl.pallas_call(
        functools.partial(_conv4_kernel, G),
        grid=(B,),
        in_specs=[
            pl.BlockSpec((1, N, CA), lambda i: (i, 0, 0)),
            pl.BlockSpec((1, C, N), lambda i: (i, 0, 0)),
            pl.BlockSpec((O, C), lambda i: (0, 0)),
            pl.BlockSpec((O,), lambda i: (0,)),
            pl.BlockSpec((O,), lambda i: (0,)),
            pl.BlockSpec((O, C), lambda i: (0, 0)),
            pl.BlockSpec((O,), lambda i: (0,)),
            pl.BlockSpec((O,), lambda i: (0,)),
        ],
        out_specs=[
            pl.BlockSpec((1, N, O // 2), lambda i: (i, 0, 0)),
            pl.BlockSpec((1, O, N), lambda i: (i, 0, 0)),
            pl.BlockSpec((1, C, N), lambda i: (i, 0, 0)),
        ],
        out_shape=[
            jax.ShapeDtypeStruct((B, N, O // 2), jnp.int32),
            jax.ShapeDtypeStruct((B, O, N), jnp.float32),
            jax.ShapeDtypeStruct((B, C, N), jnp.float32),
        ],
    )(agg, fbp, Wa, ga, ba, Wb, gb, bb)


def _head_kernel(agg4_ref, fb4_ref, x1_ref, x2_ref, x3_ref,
                 w5a_ref, g5a_ref, b5a_ref, w5b_ref, g5b_ref, b5b_ref, out_ref):
    pk = lax.bitcast_convert_type(agg4_ref[0], jnp.uint32)
    ev = lax.bitcast_convert_type(pk << 16, jnp.float32).T
    od = lax.bitcast_convert_type(pk & jnp.uint32(0xFFFF0000),
                                  jnp.float32).T
    fb4r = fb4_ref[0].reshape(128, 2, N)
    x4 = jnp.stack([fb4r[:, 0, :] + ev, fb4r[:, 1, :] + od],
                   axis=1).reshape(256, N)
    w = w5a_ref[...]
    y = (jnp.dot(w[:, 0:64], x1_ref[0], preferred_element_type=jnp.float32)
         + jnp.dot(w[:, 64:128], x2_ref[0], preferred_element_type=jnp.float32)
         + jnp.dot(w[:, 128:256], x3_ref[0], preferred_element_type=jnp.float32)
         + jnp.dot(w[:, 256:512], x4, preferred_element_type=jnp.float32))
    x5 = _gn_lrelu(y, 16, g5a_ref, b5a_ref)
    y6 = jnp.dot(w5b_ref[...], x5, preferred_element_type=jnp.float32)
    x6 = _gn_lrelu(y6, 16, g5b_ref, b5b_ref)
    out_ref[0] = x6.T


def _head(agg4, fb4, x1, x2, x3, W5a, g5a, b5a, W5b, g5b, b5b):
    return pl.pallas_call(
        _head_kernel,
        grid=(B,),
        in_specs=[
            pl.BlockSpec((1, N, 128), lambda i: (i, 0, 0)),
            pl.BlockSpec((1, 256, N), lambda i: (i, 0, 0)),
            pl.BlockSpec((1, 64, N), lambda i: (i, 0, 0)),
            pl.BlockSpec((1, 64, N), lambda i: (i, 0, 0)),
            pl.BlockSpec((1, 128, N), lambda i: (i, 0, 0)),
            pl.BlockSpec((1024, 512), lambda i: (0, 0)),
            pl.BlockSpec((1024,), lambda i: (0,)),
            pl.BlockSpec((1024,), lambda i: (0,)),
            pl.BlockSpec((512, 1024), lambda i: (0, 0)),
            pl.BlockSpec((512,), lambda i: (0,)),
            pl.BlockSpec((512,), lambda i: (0,)),
        ],
        out_specs=pl.BlockSpec((1, N, 512), lambda i: (i, 0, 0)),
        out_shape=jax.ShapeDtypeStruct((B, N, 512), jnp.float32),
    )(agg4, fb4, x1, x2, x3, W5a, g5a, b5a, W5b, g5b, b5b)


def _make_agg(k, C, CP):
    CH = C // 16
    OUT_CH = 64

    mesh = plsc.VectorSubcoreMesh(core_axis_name="c", subcore_axis_name="s",
                                  num_cores=NC, num_subcores=NS)

    def point_max(rows_buf):
        accs = [rows_buf[0, pl.ds(c * 16, 16)] for c in range(CH)]

        def body(r4, accs):
            accs = list(accs)
            r = 1 + r4 * 4
            for u in range(4):
                for c in range(CH):
                    accs[c] = jnp.maximum(
                        accs[c], rows_buf[r + u, pl.ds(c * 16, 16)])
            return tuple(accs)

        n4 = (k - 1) // 4
        accs = list(lax.fori_loop(0, n4, body, tuple(accs), unroll=False))
        for r in range(1 + n4 * 4, k):
            for c in range(CH):
                accs[c] = jnp.maximum(accs[c], rows_buf[r, pl.ds(c * 16, 16)])
        return accs

    @functools.partial(
        pl.kernel,
        out_type=jax.ShapeDtypeStruct((NROWS, C), jnp.float32),
        mesh=mesh,
        scratch_types=[
            pltpu.VMEM((ROWS_PER_W, 80), jnp.int32),
            pltpu.VMEM((k, CP), jnp.float32),
            pltpu.VMEM((k, CP), jnp.float32),
            pltpu.VMEM((OUT_CH, C), jnp.float32),
            pltpu.SemaphoreType.DMA,
            pltpu.SemaphoreType.DMA,
        ],
    )
    def agg(fa_hbm, idx_hbm, out_hbm, idx_v, buf0, buf1, out_v, sem0, sem1):
        wid = lax.axis_index("s") * NC + lax.axis_index("c")
        base = wid * ROWS_PER_W
        pltpu.sync_copy(idx_hbm.at[pl.ds(base, ROWS_PER_W), :], idx_v)

        def issue(p, buf, sem):
            pc = jnp.minimum(p, ROWS_PER_W - 1)
            pltpu.async_copy(fa_hbm.at[idx_v.at[pc, pl.ds(0, k)]], buf, sem)

        def waitbuf(buf, sem):
            pltpu.make_async_copy(fa_hbm.at[idx_v.at[0, pl.ds(0, k)]], buf,
                                  sem).wait()

        issue(0, buf0, sem0)
        issue(1, buf1, sem1)

        def flush_chunk(o, _):
            def pair(t, _):
                p0 = o * OUT_CH + 2 * t
                waitbuf(buf0, sem0)
                accs = point_max(buf0)
                for c in range(CH):
                    out_v[2 * t, pl.ds(c * 16, 16)] = accs[c]
                issue(p0 + 2, buf0, sem0)
                waitbuf(buf1, sem1)
                accs = point_max(buf1)
                for c in range(CH):
                    out_v[2 * t + 1, pl.ds(c * 16, 16)] = accs[c]
                issue(p0 + 3, buf1, sem1)
                return 0

            lax.fori_loop(0, OUT_CH // 2, pair, 0, unroll=False)
            pltpu.sync_copy(
                out_v, out_hbm.at[pl.ds(base + o * OUT_CH, OUT_CH), :])
            return 0

        lax.fori_loop(0, ROWS_PER_W // OUT_CH, flush_chunk, 0, unroll=False)
        waitbuf(buf0, sem0)
        waitbuf(buf1, sem1)

    return agg


def _make_agg_packed(k, CW):
    CH = CW // 16
    OUT_CH = 64

    mesh = plsc.VectorSubcoreMesh(core_axis_name="c", subcore_axis_name="s",
                                  num_cores=NC, num_subcores=NS)

    def point_max(rows_buf):
        accs = [plsc.bitcast(rows_buf[0, pl.ds(c * 16, 16)], jnp.bfloat16)
                for c in range(CH)]

        def body(r4, accs):
            accs = list(accs)
            r = 1 + r4 * 4
            for u in range(4):
                for c in range(CH):
                    accs[c] = jnp.maximum(accs[c], plsc.bitcast(
                        rows_buf[r + u, pl.ds(c * 16, 16)], jnp.bfloat16))
            return tuple(accs)

        n4 = (k - 1) // 4
        accs = list(lax.fori_loop(0, n4, body, tuple(accs), unroll=False))
        for r in range(1 + n4 * 4, k):
            for c in range(CH):
                accs[c] = jnp.maximum(accs[c], plsc.bitcast(
                    rows_buf[r, pl.ds(c * 16, 16)], jnp.bfloat16))
        return [plsc.bitcast(a, jnp.int32) for a in accs]

    @functools.partial(
        pl.kernel,
        out_type=jax.ShapeDtypeStruct((NROWS, CW), jnp.int32),
        mesh=mesh,
        scratch_types=[
            pltpu.VMEM((ROWS_PER_W, 80), jnp.int32),
            pltpu.VMEM((k, CW), jnp.int32),
            pltpu.VMEM((k, CW), jnp.int32),
            pltpu.VMEM((OUT_CH, CW), jnp.int32),
            pltpu.SemaphoreType.DMA,
            pltpu.SemaphoreType.DMA,
        ],
        compiler_params=pltpu.CompilerParams(needs_layout_passes=False),
    )
    def agg(fa_hbm, idx_hbm, out_hbm, idx_v, buf0, buf1, out_v, sem0, sem1):
        wid = lax.axis_index("s") * NC + lax.axis_index("c")
        base = wid * ROWS_PER_W
        pltpu.sync_copy(idx_hbm.at[pl.ds(base, ROWS_PER_W), :], idx_v)

        def issue(p, buf, sem):
            pc = jnp.minimum(p, ROWS_PER_W - 1)
            pltpu.async_copy(fa_hbm.at[idx_v.at[pc, pl.ds(0, k)]], buf, sem)

        def waitbuf(buf, sem):
            pltpu.make_async_copy(fa_hbm.at[idx_v.at[0, pl.ds(0, k)]], buf,
                                  sem).wait()

        issue(0, buf0, sem0)
        issue(1, buf1, sem1)

        def flush_chunk(o, _):
            def pair(t, _):
                p0 = o * OUT_CH + 2 * t
                waitbuf(buf0, sem0)
                accs = point_max(buf0)
                for c in range(CH):
                    out_v[2 * t, pl.ds(c * 16, 16)] = accs[c]
                issue(p0 + 2, buf0, sem0)
                waitbuf(buf1, sem1)
                accs = point_max(buf1)
                for c in range(CH):
                    out_v[2 * t + 1, pl.ds(c * 16, 16)] = accs[c]
                issue(p0 + 3, buf1, sem1)
                return 0

            lax.fori_loop(0, OUT_CH // 2, pair, 0, unroll=False)
            pltpu.sync_copy(
                out_v, out_hbm.at[pl.ds(base + o * OUT_CH, OUT_CH), :])
            return 0

        lax.fori_loop(0, ROWS_PER_W // OUT_CH, flush_chunk, 0, unroll=False)
        waitbuf(buf0, sem0)
        waitbuf(buf1, sem1)

    return agg


def _sc_agg_packed(faP, idxg, k):
    CW = faP.shape[2]
    out = _make_agg_packed(k, CW)(faP.reshape(NROWS, CW), idxg)
    return out


def _make_agg_batch(kp, GP, C, CP):
    CH = C // 16
    PB = 2 * GP

    mesh = plsc.VectorSubcoreMesh(core_axis_name="c", subcore_axis_name="s",
                                  num_cores=NC, num_subcores=NS)

    def point_max(rows_buf, g):
        r0 = g * kp
        accs = [rows_buf[r0, pl.ds(c * 16, 16)] for c in range(CH)]

        def body(r4, accs):
            accs = list(accs)
            r = r0 + 1 + r4 * 4
            for u in range(4):
                for c in range(CH):
                    accs[c] = jnp.maximum(
                        accs[c], rows_buf[r + u, pl.ds(c * 16, 16)])
            return tuple(accs)

        n4 = (kp - 1) // 4
        accs = list(lax.fori_loop(0, n4, body, tuple(accs), unroll=False))
        for r in range(r0 + 1 + n4 * 4, r0 + kp):
            for c in range(CH):
                accs[c] = jnp.maximum(accs[c], rows_buf[r, pl.ds(c * 16, 16)])
        return accs

    @functools.partial(
        pl.kernel,
        out_type=jax.ShapeDtypeStruct((NROWS, C), jnp.float32),
        mesh=mesh,
        scratch_types=[
            pltpu.VMEM((ROWS_PER_W * kp,), jnp.int32),
            pltpu.VMEM((GP * kp, CP), jnp.float32),
            pltpu.VMEM((GP * kp, CP), jnp.float32),
            pltpu.VMEM((PB * 8, C), jnp.float32),
            pltpu.SemaphoreType.DMA,
            pltpu.SemaphoreType.DMA,
        ],
    )
    def agg(fa_hbm, idx_hbm, out_hbm, idx_v, buf0, buf1, out_v, sem0, sem1):
        wid = lax.axis_index("s") * NC + lax.axis_index("c")
        base = wid * ROWS_PER_W
        pltpu.sync_copy(
            idx_hbm.at[pl.ds(base * kp, ROWS_PER_W * kp)], idx_v)

        nb = ROWS_PER_W // GP

        def issue(b, buf, sem):
            bc = jnp.minimum(b, nb - 1)
            pltpu.async_copy(fa_hbm.at[idx_v.at[pl.ds(bc * GP * kp, GP * kp)]],
                             buf, sem)

        def waitbuf(buf, sem):
            pltpu.make_async_copy(fa_hbm.at[idx_v.at[pl.ds(0, GP * kp)]], buf,
                                  sem).wait()

        issue(0, buf0, sem0)
        issue(1, buf1, sem1)

        def flush_chunk(o, _):
            def pairb(t, _):
                b0 = o * 16 + 2 * t
                waitbuf(buf0, sem0)
                for g in range(GP):
                    accs = point_max(buf0, g)
                    for c in range(CH):
                        out_v[(2 * t) * GP + g, pl.ds(c * 16, 16)] = accs[c]
                issue(b0 + 2, buf0, sem0)
                waitbuf(buf1, sem1)
                for g in range(GP):
                    accs = point_max(buf1, g)
                    for c in range(CH):
                        out_v[(2 * t + 1) * GP + g,
                              pl.ds(c * 16, 16)] = accs[c]
                issue(b0 + 3, buf1, sem1)
                return 0

            lax.fori_loop(0, 8, pairb, 0, unroll=False)
            pltpu.sync_copy(
                out_v,
                out_hbm.at[pl.ds(base + o * PB * 8, PB * 8), :])
            return 0

        lax.fori_loop(0, ROWS_PER_W // (PB * 8), flush_chunk, 0,
                      unroll=False)
        waitbuf(buf0, sem0)
        waitbuf(buf1, sem1)

    return agg


def _sc_agg_batch(faT, idxf, kp, GP, C):
    CP = faT.shape[2]
    return _make_agg_batch(kp, GP, C, CP)(faT.reshape(NROWS, CP), idxf)


def _sc_agg(faT, idxg, k, C):
    CP = faT.shape[2]
    return _make_agg(k, C, CP)(faT.reshape(NROWS, CP), idxg)


_NEG = -3.0e38


def _ce(mk, mp, i, j):
    m = mk[i] >= mk[j]
    hik = jnp.where(m, mk[i], mk[j])
    lok = jnp.where(m, mk[j], mk[i])
    hip = jnp.where(m, mp[i], mp[j])
    lop = jnp.where(m, mp[j], mp[i])
    return hik, lok, hip, lop


def _merge_pair(Lk, Lp, wlk, wlp, whk, whp):
    neg = jnp.full((16,), _NEG, jnp.float32)
    zero = jnp.zeros((16,), jnp.int32)
    mk = [Lk[0], Lk[1], Lk[2], Lk[3], Lk[4], neg, wlk, whk]
    mp = [Lp[0], Lp[1], Lp[2], Lp[3], Lp[4], zero, wlp, whp]
    for (i, j) in ((0, 4), (2, 6), (3, 7)):
        mk[i], mk[j], mp[i], mp[j] = _ce(mk, mp, i, j)
    for (i, j) in ((0, 2), (1, 3), (4, 6)):
        mk[i], mk[j], mp[i], mp[j] = _ce(mk, mp, i, j)
    mk[5], mp[5] = mk[7], mp[7]
    for (i, j) in ((0, 1), (2, 3), (4, 5)):
        mk[i], mk[j], mp[i], mp[j] = _ce(mk, mp, i, j)
    outk, outp = [], []
    for i in range(5):
        ks, ps = plsc.sort_key_val(mk[i], mp[i], descending=True)
        outk.append(ks)
        outp.append(ps)
    return outk, outp


def _make_topk():
    OUT_CH = 64
    CAP = N + 32

    mesh = plsc.VectorSubcoreMesh(core_axis_name="c", subcore_axis_name="s",
                                  num_cores=NC, num_subcores=NS)

    @functools.partial(
        pl.kernel,
        out_type=[
            jax.ShapeDtypeStruct((NROWS, 80), jnp.int32),
            jax.ShapeDtypeStruct((NROWS * 32,), jnp.int32),
            jax.ShapeDtypeStruct((NROWS * 48,), jnp.int32),
        ],
        mesh=mesh,
        scratch_types=[
            pltpu.VMEM((N,), jnp.float32),
            pltpu.VMEM((N,), jnp.float32),
            pltpu.VMEM((256,), jnp.float32),
            pltpu.VMEM((256,), jnp.float32),
            pltpu.VMEM((CAP,), jnp.float32),
            pltpu.VMEM((CAP,), jnp.int32),
            pltpu.VMEM((OUT_CH, 80), jnp.int32),
            pltpu.VMEM((OUT_CH * 32,), jnp.int32),
            pltpu.VMEM((OUT_CH * 48,), jnp.int32),
            pltpu.SemaphoreType.DMA,
            pltpu.SemaphoreType.DMA,
            pltpu.SemaphoreType.DMA,
            pltpu.SemaphoreType.DMA,
        ],
        compiler_params=pltpu.CompilerParams(needs_layout_passes=False),
    )
    def topk(pd_hbm, cm_hbm, out_hbm, out1_hbm, out2_hbm, row0, row1,
             cmb0, cmb1, cand_v, cand_i, out_v, out_v1, out_v2,
             semr0, semr1, semc0, semc1):
        wid = lax.axis_index("s") * NC + lax.axis_index("c")
        base = wid * ROWS_PER_W
        joff = (base // N) * N

        iota = lax.broadcasted_iota(jnp.int32, (16,), 0)

        def issue(p, rowb, cmb, semr, semc):
            pc = jnp.minimum(p, ROWS_PER_W - 1)
            pltpu.async_copy(pd_hbm.at[base + pc, :], rowb, semr)
            pltpu.async_copy(cm_hbm.at[base + pc, :], cmb, semc)

        def waitb(rowb, cmb, semr, semc):
            pltpu.make_async_copy(pd_hbm.at[base, :], rowb, semr).wait()
            pltpu.make_async_copy(cm_hbm.at[base, :], cmb, semc).wait()

        def process(p, rowb, cmb):
            neg = jnp.full((16,), _NEG, jnp.float32)
            r = [neg, neg, neg, neg, neg]
            for i in range(16):
                v = cmb[pl.ds(i * 16, 16)]
                for s in range(5):
                    hi = jnp.maximum(r[s], v)
                    v = jnp.minimum(r[s], v)
                    r[s] = hi
            t = jnp.min(r[4])
            tv = jnp.full((16,), t, jnp.float32)

            def compact(jb, off):
                for u in range(8):
                    j0 = (jb * 8 + u) * 16
                    v = rowb[pl.ds(j0, 16)]
                    m = v >= tv
                    plsc.store_compressed(cand_v.at[pl.ds(off, 16)], v, mask=m)
                    plsc.store_compressed(cand_i.at[pl.ds(off, 16)],
                                          iota + j0, mask=m)
                    off = off + jnp.sum(m.astype(jnp.int32))
                return off

            off = lax.fori_loop(0, 16, compact, jnp.int32(0), unroll=False)
            cand_v[pl.ds(off, 16)] = neg
            cand_v[pl.ds(off + 16, 16)] = neg

            zero = jnp.zeros((16,), jnp.int32)
            init = (neg, neg, neg, neg, neg, zero, zero, zero, zero, zero)

            def sel(i2, carry):
                Lk = list(carry[0:5])
                Lp = list(carry[5:10])
                v1 = cand_v[pl.ds(i2 * 32, 16)]
                p1 = cand_i[pl.ds(i2 * 32, 16)]
                v2 = cand_v[pl.ds(i2 * 32 + 16, 16)]
                p2 = cand_i[pl.ds(i2 * 32 + 16, 16)]
                lmin = jnp.min(Lk[4])

                def do_merge(_):
                    s1k, s1p = plsc.sort_key_val(v1, p1, descending=False)
                    s2k, s2p = plsc.sort_key_val(v2, p2, descending=False)
                    r2k = lax.rev(s2k, (0,))
                    r2p = lax.rev(s2p, (0,))
                    m = s1k >= r2k
                    hk = jnp.where(m, s1k, r2k)
                    hp = jnp.where(m, s1p, r2p)
                    lk = jnp.where(m, r2k, s1k)
                    lp = jnp.where(m, r2p, s1p)
                    wlk, wlp = plsc.sort_key_val(lk, lp, descending=False)
                    whk, whp = plsc.sort_key_val(hk, hp, descending=False)
                    nk, np_ = _merge_pair(Lk, Lp, wlk, wlp, whk, whp)
                    return tuple(nk) + tuple(np_)

                def skip(_):
                    return tuple(Lk) + tuple(Lp)

                cnt = jnp.sum((jnp.maximum(v1, v2)
                               >= jnp.full((16,), lmin)).astype(jnp.int32))
                return lax.cond(cnt > 0, do_merge, skip, 0)

            nv2 = (off + 31) // 32
            fin = lax.fori_loop(0, nv2, sel, init, unroll=False)

            prow = p % OUT_CH
            gl = [fin[5 + g] + joff for g in range(5)]
            for g in range(5):
                out_v[prow, pl.ds(g * 16, 16)] = gl[g]
            selfv = jnp.full((16,), 0, jnp.int32) + (base + p)
            out_v1[pl.ds(prow * 32, 16)] = gl[0]
            out_v1[pl.ds(prow * 32 + 16, 16)] = jnp.where(iota < 4, gl[1],
                                                          selfv)
            out_v2[pl.ds(prow * 48, 16)] = gl[0]
            out_v2[pl.ds(prow * 48 + 16, 16)] = gl[1]
            out_v2[pl.ds(prow * 48 + 32, 16)] = jnp.where(iota < 8, gl[2],
                                                          selfv)

        issue(0, row0, cmb0, semr0, semc0)
        issue(1, row1, cmb1, semr1, semc1)

        def flush_chunk(o, _):
            def pair(tt, _):
                p0 = o * OUT_CH + 2 * tt
                waitb(row0, cmb0, semr0, semc0)
                process(p0, row0, cmb0)
                issue(p0 + 2, row0, cmb0, semr0, semc0)
                waitb(row1, cmb1, semr1, semc1)
                process(p0 + 1, row1, cmb1)
                issue(p0 + 3, row1, cmb1, semr1, semc1)
                return 0

            lax.fori_loop(0, OUT_CH // 2, pair, 0, unroll=False)
            pltpu.sync_copy(
                out_v, out_hbm.at[pl.ds(base + o * OUT_CH, OUT_CH), :])
            pltpu.sync_copy(
                out_v1,
                out1_hbm.at[pl.ds((base + o * OUT_CH) * 32, OUT_CH * 32)])
            pltpu.sync_copy(
                out_v2,
                out2_hbm.at[pl.ds((base + o * OUT_CH) * 48, OUT_CH * 48)])
            return 0

        lax.fori_loop(0, ROWS_PER_W // OUT_CH, flush_chunk, 0, unroll=False)
        waitb(row0, cmb0, semr0, semc0)
        waitb(row1, cmb1, semr1, semc1)

    return topk


def _sc_topk(pd, cm):
    return _make_topk()(pd.reshape(NROWS, N), cm.reshape(NROWS, 256))


def kernel(x, W1a, g1a, b1a, W1b, g1b, b1b, W2a, g2a, b2a, W2b, g2b, b2b,
           W3a, g3a, b3a, W3b, g3b, b3b, W4a, g4a, b4a, W4b, g4b, b4b,
           W5a, g5a, b5a, W5b, g5b, b5b):
    xt = jnp.transpose(x, (0, 2, 1))
    x8 = jnp.pad(x, ((0, 0), (0, 0), (0, 5)))
    xt8 = jnp.pad(xt, ((0, 0), (0, 5), (0, 0)))
    norms = jnp.sum(x * x, axis=-1)[:, None, :]
    Wa8 = jnp.pad(W1a, ((0, 0), (0, 5)))
    Wb8 = jnp.pad(W1b, ((0, 0), (0, 5)))

    pd, cm = _pd_chunkmax(x8, xt8, norms)
    idxg, idx1f, idx2f = _sc_topk(pd, cm)

    fa1T, fb1 = _conv1(xt8, Wa8, g1a, b1a, Wb8, g1b, b1b, 128)
    agg1 = _sc_agg_batch(fa1T, idx1f, 32, 4, 64).reshape(B, N, 64)
    fa2T, fb2, x1 = _conv_stage(agg1, fb1, W2a, g2a, b2a, W2b, g2b, b2b, 8, 128)
    agg2 = _sc_agg_batch(fa2T, idx2f, 48, 2, 64).reshape(B, N, 64)
    fa3T, fb3, x2 = _conv_stage(agg2, fb2, W3a, g3a, b3a, W3b, g3b, b3b, 8, 128)
    agg3 = _sc_agg(fa3T, idxg, K + 2 * P, 128).reshape(B, N, 128)
    fa4P, fb4, x3 = _conv_stage4(agg3, fb3, W4a, g4a, b4a, W4b, g4b, b4b, 16)
    agg4 = _sc_agg_packed(fa4P, idxg, K + 3 * P).reshape(B, N, 128)

    return _head(agg4, fb4, x1, x2, x3, W5a, g5a, b5a, W5b, g5b, b5b)


# speedup vs baseline: 26.6331x; 1.0479x over previous
import functools

import jax
import jax.numpy as jnp
from jax import lax
from jax.experimental import pallas as pl
from jax.experimental.pallas import tpu as pltpu
from jax.experimental.pallas import tpu_sc as plsc

K = 20
P = 20
B = 8
N = 2048
NROWS = B * N
NC = 2
NS = 16
NW = NC * NS
ROWS_PER_W = NROWS // NW


def _pd_kernel(x8_ref, xt8_ref, norms_ref, pd_ref, cm_ref):
    xb = x8_ref[0]
    xbt = xt8_ref[0]
    g = jnp.dot(xb, xbt, preferred_element_type=jnp.float32)
    pd = 2.0 * g - norms_ref[0]
    pd_ref[0] = pd
    cm = pd[:, 0:256]
    for c in range(1, 8):
        cm = jnp.maximum(cm, pd[:, c * 256:(c + 1) * 256])
    cm_ref[0] = cm


def _pd_chunkmax(x8, xt8, norms):
    return pl.pallas_call(
        _pd_kernel,
        grid=(B,),
        in_specs=[
            pl.BlockSpec((1, N, 8), lambda i: (i, 0, 0)),
            pl.BlockSpec((1, 8, N), lambda i: (i, 0, 0)),
            pl.BlockSpec((1, 1, N), lambda i: (i, 0, 0)),
        ],
        out_specs=[
            pl.BlockSpec((1, N, N), lambda i: (i, 0, 0)),
            pl.BlockSpec((1, N, 256), lambda i: (i, 0, 0)),
        ],
        out_shape=[
            jax.ShapeDtypeStruct((B, N, N), jnp.float32),
            jax.ShapeDtypeStruct((B, N, 256), jnp.float32),
        ],
    )(x8, xt8, norms)


def _gn_lrelu(y, G, g_ref, b_ref):
    O = y.shape[0]
    yg = y.reshape(G, O // G, N)
    m = jnp.mean(yg, axis=(1, 2), keepdims=True)
    v = jnp.mean((yg - m) ** 2, axis=(1, 2), keepdims=True)
    yg = (yg - m) * lax.rsqrt(v + 1e-5)
    y = yg.reshape(O, N)
    y = y * g_ref[...][:, None] + b_ref[...][:, None]
    return jnp.where(y >= 0, y, 0.2 * y)


def _block_pair(x_s, wa_ref, ga_ref, ba_ref, wb_ref, gb_ref, bb_ref, G):
    fa = _gn_lrelu(jnp.dot(wa_ref[...], x_s, preferred_element_type=jnp.float32),
                   G, ga_ref, ba_ref)
    fb = _gn_lrelu(jnp.dot(wb_ref[...], x_s, preferred_element_type=jnp.float32),
                   G, gb_ref, bb_ref)
    return fa, fb


def _conv1_kernel(xt8_ref, wa_ref, ga_ref, ba_ref, wb_ref, gb_ref, bb_ref,
                  faT_ref, fb_ref):
    x_s = xt8_ref[0]
    fa, fb = _block_pair(x_s, wa_ref, ga_ref, ba_ref, wb_ref, gb_ref, bb_ref, 8)
    faT = fa.T
    O = faT.shape[1]
    CP = faT_ref.shape[2]
    if CP > O:
        faT = jnp.concatenate(
            [faT, jnp.zeros((faT.shape[0], CP - O), jnp.float32)], axis=1)
    faT_ref[0] = faT
    fb_ref[0] = fb


def _conv1(xt8, Wa8, ga, ba, Wb8, gb, bb, CP):
    O = Wa8.shape[0]
    return pl.pallas_call(
        _conv1_kernel,
        grid=(B,),
        in_specs=[
            pl.BlockSpec((1, 8, N), lambda i: (i, 0, 0)),
            pl.BlockSpec(Wa8.shape, lambda i: (0, 0)),
            pl.BlockSpec((O,), lambda i: (0,)),
            pl.BlockSpec((O,), lambda i: (0,)),
            pl.BlockSpec(Wb8.shape, lambda i: (0, 0)),
            pl.BlockSpec((O,), lambda i: (0,)),
            pl.BlockSpec((O,), lambda i: (0,)),
        ],
        out_specs=[
            pl.BlockSpec((1, N, CP), lambda i: (i, 0, 0)),
            pl.BlockSpec((1, O, N), lambda i: (i, 0, 0)),
        ],
        out_shape=[
            jax.ShapeDtypeStruct((B, N, CP), jnp.float32),
            jax.ShapeDtypeStruct((B, O, N), jnp.float32),
        ],
    )(xt8, Wa8, ga, ba, Wb8, gb, bb)


def _conv_stage_kernel(G, agg_ref, fbp_ref, wa_ref, ga_ref, ba_ref,
                       wb_ref, gb_ref, bb_ref, faT_ref, fb_ref, xprev_ref):
    x_s = fbp_ref[0] + agg_ref[0].T
    xprev_ref[0] = x_s
    fa, fb = _block_pair(x_s, wa_ref, ga_ref, ba_ref, wb_ref, gb_ref, bb_ref, G)
    faT = fa.T
    O = faT.shape[1]
    CP = faT_ref.shape[2]
    if CP > O:
        faT = jnp.concatenate(
            [faT, jnp.zeros((faT.shape[0], CP - O), jnp.float32)], axis=1)
    faT_ref[0] = faT
    fb_ref[0] = fb


def _conv_stage(agg, fbp, Wa, ga, ba, Wb, gb, bb, G, CP):
    C = Wa.shape[1]
    CA = agg.shape[2]
    O = Wa.shape[0]
    return pl.pallas_call(
        functools.partial(_conv_stage_kernel, G),
        grid=(B,),
        in_specs=[
            pl.BlockSpec((1, N, CA), lambda i: (i, 0, 0)),
            pl.BlockSpec((1, C, N), lambda i: (i, 0, 0)),
            pl.BlockSpec((O, C), lambda i: (0, 0)),
            pl.BlockSpec((O,), lambda i: (0,)),
            pl.BlockSpec((O,), lambda i: (0,)),
            pl.BlockSpec((O, C), lambda i: (0, 0)),
            pl.BlockSpec((O,), lambda i: (0,)),
            pl.BlockSpec((O,), lambda i: (0,)),
        ],
        out_specs=[
            pl.BlockSpec((1, N, CP), lambda i: (i, 0, 0)),
            pl.BlockSpec((1, O, N), lambda i: (i, 0, 0)),
            pl.BlockSpec((1, C, N), lambda i: (i, 0, 0)),
        ],
        out_shape=[
            jax.ShapeDtypeStruct((B, N, CP), jnp.float32),
            jax.ShapeDtypeStruct((B, O, N), jnp.float32),
            jax.ShapeDtypeStruct((B, C, N), jnp.float32),
        ],
    )(agg, fbp, Wa, ga, ba, Wb, gb, bb)


def _conv4_kernel(G, agg_ref, fbp_ref, wa_ref, ga_ref, ba_ref,
                  wb_ref, gb_ref, bb_ref, faP_ref, fb_ref, xprev_ref):
    x_s = fbp_ref[0] + agg_ref[0].T
    xprev_ref[0] = x_s
    fa, fb = _block_pair(x_s, wa_ref, ga_ref, ba_ref, wb_ref, gb_ref, bb_ref, G)
    u = lax.bitcast_convert_type(fa, jnp.uint32)
    bf = (u + jnp.uint32(0x7FFF) + ((u >> 16) & jnp.uint32(1))) >> 16
    O = fa.shape[0]
    bfp = bf.reshape(O // 2, 2, N)
    packed = bfp[:, 0, :] | (bfp[:, 1, :] << 16)
    faP_ref[0] = lax.bitcast_convert_type(packed.T, jnp.int32)
    fb_ref[0] = fb


def _conv_stage4(agg, fbp, Wa, ga, ba, Wb, gb, bb, G):
    C = Wa.shape[1]
    CA = agg.shape[2]
    O = Wa.shape[0]
    return pl.pallas_call(
        functools.partial(_conv4_kernel, G),
        grid=(B,),
        in_specs=[
            pl.BlockSpec((1, N, CA), lambda i: (i, 0, 0)),
            pl.BlockSpec((1, C, N), lambda i: (i, 0, 0)),
            pl.BlockSpec((O, C), lambda i: (0, 0)),
            pl.BlockSpec((O,), lambda i: (0,)),
            pl.BlockSpec((O,), lambda i: (0,)),
            pl.BlockSpec((O, C), lambda i: (0, 0)),
            pl.BlockSpec((O,), lambda i: (0,)),
            pl.BlockSpec((O,), lambda i: (0,)),
        ],
        out_specs=[
            pl.BlockSpec((1, N, O // 2), lambda i: (i, 0, 0)),
            pl.BlockSpec((1, O, N), lambda i: (i, 0, 0)),
            pl.BlockSpec((1, C, N), lambda i: (i, 0, 0)),
        ],
        out_shape=[
            jax.ShapeDtypeStruct((B, N, O // 2), jnp.int32),
            jax.ShapeDtypeStruct((B, O, N), jnp.float32),
            jax.ShapeDtypeStruct((B, C, N), jnp.float32),
        ],
    )(agg, fbp, Wa, ga, ba, Wb, gb, bb)


def _head_kernel(agg4_ref, fb4_ref, x1_ref, x2_ref, x3_ref,
                 w5a_ref, g5a_ref, b5a_ref, w5b_ref, g5b_ref, b5b_ref, out_ref):
    pk = lax.bitcast_convert_type(agg4_ref[0], jnp.uint32)
    ev = lax.bitcast_convert_type(pk << 16, jnp.float32).T
    od = lax.bitcast_convert_type(pk & jnp.uint32(0xFFFF0000),
                                  jnp.float32).T
    fb4r = fb4_ref[0].reshape(128, 2, N)
    x4 = jnp.stack([fb4r[:, 0, :] + ev, fb4r[:, 1, :] + od],
                   axis=1).reshape(256, N)
    w = w5a_ref[...].astype(jnp.bfloat16)
    bf = jnp.bfloat16
    y = (jnp.dot(w[:, 0:64], x1_ref[0].astype(bf),
                 preferred_element_type=jnp.float32)
         + jnp.dot(w[:, 64:128], x2_ref[0].astype(bf),
                   preferred_element_type=jnp.float32)
         + jnp.dot(w[:, 128:256], x3_ref[0].astype(bf),
                   preferred_element_type=jnp.float32)
         + jnp.dot(w[:, 256:512], x4.astype(bf),
                   preferred_element_type=jnp.float32))
    x5 = _gn_lrelu(y, 16, g5a_ref, b5a_ref)
    y6 = jnp.dot(w5b_ref[...].astype(bf), x5.astype(bf),
                 preferred_element_type=jnp.float32)
    x6 = _gn_lrelu(y6, 16, g5b_ref, b5b_ref)
    out_ref[0] = x6.T


def _head(agg4, fb4, x1, x2, x3, W5a, g5a, b5a, W5b, g5b, b5b):
    return pl.pallas_call(
        _head_kernel,
        grid=(B,),
        in_specs=[
            pl.BlockSpec((1, N, 128), lambda i: (i, 0, 0)),
            pl.BlockSpec((1, 256, N), lambda i: (i, 0, 0)),
            pl.BlockSpec((1, 64, N), lambda i: (i, 0, 0)),
            pl.BlockSpec((1, 64, N), lambda i: (i, 0, 0)),
            pl.BlockSpec((1, 128, N), lambda i: (i, 0, 0)),
            pl.BlockSpec((1024, 512), lambda i: (0, 0)),
            pl.BlockSpec((1024,), lambda i: (0,)),
            pl.BlockSpec((1024,), lambda i: (0,)),
            pl.BlockSpec((512, 1024), lambda i: (0, 0)),
            pl.BlockSpec((512,), lambda i: (0,)),
            pl.BlockSpec((512,), lambda i: (0,)),
        ],
        out_specs=pl.BlockSpec((1, N, 512), lambda i: (i, 0, 0)),
        out_shape=jax.ShapeDtypeStruct((B, N, 512), jnp.float32),
    )(agg4, fb4, x1, x2, x3, W5a, g5a, b5a, W5b, g5b, b5b)


def _make_agg(k, C, CP):
    CH = C // 16
    OUT_CH = 64

    mesh = plsc.VectorSubcoreMesh(core_axis_name="c", subcore_axis_name="s",
                                  num_cores=NC, num_subcores=NS)

    def point_max(rows_buf):
        accs = [rows_buf[0, pl.ds(c * 16, 16)] for c in range(CH)]

        def body(r4, accs):
            accs = list(accs)
            r = 1 + r4 * 4
            for u in range(4):
                for c in range(CH):
                    accs[c] = jnp.maximum(
                        accs[c], rows_buf[r + u, pl.ds(c * 16, 16)])
            return tuple(accs)

        n4 = (k - 1) // 4
        accs = list(lax.fori_loop(0, n4, body, tuple(accs), unroll=False))
        for r in range(1 + n4 * 4, k):
            for c in range(CH):
                accs[c] = jnp.maximum(accs[c], rows_buf[r, pl.ds(c * 16, 16)])
        return accs

    @functools.partial(
        pl.kernel,
        out_type=jax.ShapeDtypeStruct((NROWS, C), jnp.float32),
        mesh=mesh,
        scratch_types=[
            pltpu.VMEM((ROWS_PER_W, 80), jnp.int32),
            pltpu.VMEM((k, CP), jnp.float32),
            pltpu.VMEM((k, CP), jnp.float32),
            pltpu.VMEM((OUT_CH, C), jnp.float32),
            pltpu.SemaphoreType.DMA,
            pltpu.SemaphoreType.DMA,
        ],
    )
    def agg(fa_hbm, idx_hbm, out_hbm, idx_v, buf0, buf1, out_v, sem0, sem1):
        wid = lax.axis_index("s") * NC + lax.axis_index("c")
        base = wid * ROWS_PER_W
        pltpu.sync_copy(idx_hbm.at[pl.ds(base, ROWS_PER_W), :], idx_v)

        def issue(p, buf, sem):
            pc = jnp.minimum(p, ROWS_PER_W - 1)
            pltpu.async_copy(fa_hbm.at[idx_v.at[pc, pl.ds(0, k)]], buf, sem)

        def waitbuf(buf, sem):
            pltpu.make_async_copy(fa_hbm.at[idx_v.at[0, pl.ds(0, k)]], buf,
                                  sem).wait()

        issue(0, buf0, sem0)
        issue(1, buf1, sem1)

        def flush_chunk(o, _):
            def pair(t, _):
                p0 = o * OUT_CH + 2 * t
                waitbuf(buf0, sem0)
                accs = point_max(buf0)
                for c in range(CH):
                    out_v[2 * t, pl.ds(c * 16, 16)] = accs[c]
                issue(p0 + 2, buf0, sem0)
                waitbuf(buf1, sem1)
                accs = point_max(buf1)
                for c in range(CH):
                    out_v[2 * t + 1, pl.ds(c * 16, 16)] = accs[c]
                issue(p0 + 3, buf1, sem1)
                return 0

            lax.fori_loop(0, OUT_CH // 2, pair, 0, unroll=False)
            pltpu.sync_copy(
                out_v, out_hbm.at[pl.ds(base + o * OUT_CH, OUT_CH), :])
            return 0

        lax.fori_loop(0, ROWS_PER_W // OUT_CH, flush_chunk, 0, unroll=False)
        waitbuf(buf0, sem0)
        waitbuf(buf1, sem1)

    return agg


def _make_agg_packed(k, CW):
    CH = CW // 16
    OUT_CH = 64

    mesh = plsc.VectorSubcoreMesh(core_axis_name="c", subcore_axis_name="s",
                                  num_cores=NC, num_subcores=NS)

    def point_max(rows_buf):
        accs = [plsc.bitcast(rows_buf[0, pl.ds(c * 16, 16)], jnp.bfloat16)
                for c in range(CH)]

        def body(r4, accs):
            accs = list(accs)
            r = 1 + r4 * 4
            for u in range(4):
                for c in range(CH):
                    accs[c] = jnp.maximum(accs[c], plsc.bitcast(
                        rows_buf[r + u, pl.ds(c * 16, 16)], jnp.bfloat16))
            return tuple(accs)

        n4 = (k - 1) // 4
        accs = list(lax.fori_loop(0, n4, body, tuple(accs), unroll=False))
        for r in range(1 + n4 * 4, k):
            for c in range(CH):
                accs[c] = jnp.maximum(accs[c], plsc.bitcast(
                    rows_buf[r, pl.ds(c * 16, 16)], jnp.bfloat16))
        return [plsc.bitcast(a, jnp.int32) for a in accs]

    @functools.partial(
        pl.kernel,
        out_type=jax.ShapeDtypeStruct((NROWS, CW), jnp.int32),
        mesh=mesh,
        scratch_types=[
            pltpu.VMEM((ROWS_PER_W, 80), jnp.int32),
            pltpu.VMEM((k, CW), jnp.int32),
            pltpu.VMEM((k, CW), jnp.int32),
            pltpu.VMEM((OUT_CH, CW), jnp.int32),
            pltpu.SemaphoreType.DMA,
            pltpu.SemaphoreType.DMA,
        ],
        compiler_params=pltpu.CompilerParams(needs_layout_passes=False),
    )
    def agg(fa_hbm, idx_hbm, out_hbm, idx_v, buf0, buf1, out_v, sem0, sem1):
        wid = lax.axis_index("s") * NC + lax.axis_index("c")
        base = wid * ROWS_PER_W
        pltpu.sync_copy(idx_hbm.at[pl.ds(base, ROWS_PER_W), :], idx_v)

        def issue(p, buf, sem):
            pc = jnp.minimum(p, ROWS_PER_W - 1)
            pltpu.async_copy(fa_hbm.at[idx_v.at[pc, pl.ds(0, k)]], buf, sem)

        def waitbuf(buf, sem):
            pltpu.make_async_copy(fa_hbm.at[idx_v.at[0, pl.ds(0, k)]], buf,
                                  sem).wait()

        issue(0, buf0, sem0)
        issue(1, buf1, sem1)

        def flush_chunk(o, _):
            def pair(t, _):
                p0 = o * OUT_CH + 2 * t
                waitbuf(buf0, sem0)
                accs = point_max(buf0)
                for c in range(CH):
                    out_v[2 * t, pl.ds(c * 16, 16)] = accs[c]
                issue(p0 + 2, buf0, sem0)
                waitbuf(buf1, sem1)
                accs = point_max(buf1)
                for c in range(CH):
                    out_v[2 * t + 1, pl.ds(c * 16, 16)] = accs[c]
                issue(p0 + 3, buf1, sem1)
                return 0

            lax.fori_loop(0, OUT_CH // 2, pair, 0, unroll=False)
            pltpu.sync_copy(
                out_v, out_hbm.at[pl.ds(base + o * OUT_CH, OUT_CH), :])
            return 0

        lax.fori_loop(0, ROWS_PER_W // OUT_CH, flush_chunk, 0, unroll=False)
        waitbuf(buf0, sem0)
        waitbuf(buf1, sem1)

    return agg


def _sc_agg_packed(faP, idxg, k):
    CW = faP.shape[2]
    out = _make_agg_packed(k, CW)(faP.reshape(NROWS, CW), idxg)
    return out


def _make_agg_batch(kp, GP, C, CP):
    CH = C // 16
    PB = 2 * GP

    mesh = plsc.VectorSubcoreMesh(core_axis_name="c", subcore_axis_name="s",
                                  num_cores=NC, num_subcores=NS)

    def point_max(rows_buf, g):
        r0 = g * kp
        accs = [rows_buf[r0, pl.ds(c * 16, 16)] for c in range(CH)]

        def body(r4, accs):
            accs = list(accs)
            r = r0 + 1 + r4 * 4
            for u in range(4):
                for c in range(CH):
                    accs[c] = jnp.maximum(
                        accs[c], rows_buf[r + u, pl.ds(c * 16, 16)])
            return tuple(accs)

        n4 = (kp - 1) // 4
        accs = list(lax.fori_loop(0, n4, body, tuple(accs), unroll=False))
        for r in range(r0 + 1 + n4 * 4, r0 + kp):
            for c in range(CH):
                accs[c] = jnp.maximum(accs[c], rows_buf[r, pl.ds(c * 16, 16)])
        return accs

    @functools.partial(
        pl.kernel,
        out_type=jax.ShapeDtypeStruct((NROWS, C), jnp.float32),
        mesh=mesh,
        scratch_types=[
            pltpu.VMEM((ROWS_PER_W * kp,), jnp.int32),
            pltpu.VMEM((GP * kp, CP), jnp.float32),
            pltpu.VMEM((GP * kp, CP), jnp.float32),
            pltpu.VMEM((PB * 8, C), jnp.float32),
            pltpu.SemaphoreType.DMA,
            pltpu.SemaphoreType.DMA,
        ],
    )
    def agg(fa_hbm, idx_hbm, out_hbm, idx_v, buf0, buf1, out_v, sem0, sem1):
        wid = lax.axis_index("s") * NC + lax.axis_index("c")
        base = wid * ROWS_PER_W
        pltpu.sync_copy(
            idx_hbm.at[pl.ds(base * kp, ROWS_PER_W * kp)], idx_v)

        nb = ROWS_PER_W // GP

        def issue(b, buf, sem):
            bc = jnp.minimum(b, nb - 1)
            pltpu.async_copy(fa_hbm.at[idx_v.at[pl.ds(bc * GP * kp, GP * kp)]],
                             buf, sem)

        def waitbuf(buf, sem):
            pltpu.make_async_copy(fa_hbm.at[idx_v.at[pl.ds(0, GP * kp)]], buf,
                                  sem).wait()

        issue(0, buf0, sem0)
        issue(1, buf1, sem1)

        def flush_chunk(o, _):
            def pairb(t, _):
                b0 = o * 16 + 2 * t
                waitbuf(buf0, sem0)
                for g in range(GP):
                    accs = point_max(buf0, g)
                    for c in range(CH):
                        out_v[(2 * t) * GP + g, pl.ds(c * 16, 16)] = accs[c]
                issue(b0 + 2, buf0, sem0)
                waitbuf(buf1, sem1)
                for g in range(GP):
                    accs = point_max(buf1, g)
                    for c in range(CH):
                        out_v[(2 * t + 1) * GP + g,
                              pl.ds(c * 16, 16)] = accs[c]
                issue(b0 + 3, buf1, sem1)
                return 0

            lax.fori_loop(0, 8, pairb, 0, unroll=False)
            pltpu.sync_copy(
                out_v,
                out_hbm.at[pl.ds(base + o * PB * 8, PB * 8), :])
            return 0

        lax.fori_loop(0, ROWS_PER_W // (PB * 8), flush_chunk, 0,
                      unroll=False)
        waitbuf(buf0, sem0)
        waitbuf(buf1, sem1)

    return agg


def _sc_agg_batch(faT, idxf, kp, GP, C):
    CP = faT.shape[2]
    return _make_agg_batch(kp, GP, C, CP)(faT.reshape(NROWS, CP), idxf)


def _sc_agg(faT, idxg, k, C):
    CP = faT.shape[2]
    return _make_agg(k, C, CP)(faT.reshape(NROWS, CP), idxg)


_NEG = -3.0e38


def _lane_min_splat(x):
    for sh in (8, 4, 2, 1):
        perm = (lax.broadcasted_iota(jnp.int32, (16,), 0) + sh) & 15
        x = jnp.minimum(x, x.at[perm].get(mode="promise_in_bounds"))
    return x


def _ce(mk, mp, i, j):
    m = mk[i] >= mk[j]
    hik = jnp.where(m, mk[i], mk[j])
    lok = jnp.where(m, mk[j], mk[i])
    hip = jnp.where(m, mp[i], mp[j])
    lop = jnp.where(m, mp[j], mp[i])
    return hik, lok, hip, lop


def _merge_pair(Lk, Lp, wlk, wlp, whk, whp):
    neg = jnp.full((16,), _NEG, jnp.float32)
    zero = jnp.zeros((16,), jnp.int32)
    mk = [Lk[0], Lk[1], Lk[2], Lk[3], Lk[4], neg, wlk, whk]
    mp = [Lp[0], Lp[1], Lp[2], Lp[3], Lp[4], zero, wlp, whp]
    for (i, j) in ((0, 4), (2, 6), (3, 7)):
        mk[i], mk[j], mp[i], mp[j] = _ce(mk, mp, i, j)
    for (i, j) in ((0, 2), (1, 3), (4, 6)):
        mk[i], mk[j], mp[i], mp[j] = _ce(mk, mp, i, j)
    mk[5], mp[5] = mk[7], mp[7]
    for (i, j) in ((0, 1), (2, 3), (4, 5)):
        mk[i], mk[j], mp[i], mp[j] = _ce(mk, mp, i, j)
    outk, outp = [], []
    for i in range(5):
        ks, ps = plsc.sort_key_val(mk[i], mp[i], descending=True)
        outk.append(ks)
        outp.append(ps)
    return outk, outp


def _make_topk():
    OUT_CH = 64
    CAP = N + 32

    mesh = plsc.VectorSubcoreMesh(core_axis_name="c", subcore_axis_name="s",
                                  num_cores=NC, num_subcores=NS)

    @functools.partial(
        pl.kernel,
        out_type=[
            jax.ShapeDtypeStruct((NROWS, 80), jnp.int32),
            jax.ShapeDtypeStruct((NROWS * 32,), jnp.int32),
            jax.ShapeDtypeStruct((NROWS * 48,), jnp.int32),
        ],
        mesh=mesh,
        scratch_types=[
            pltpu.VMEM((N,), jnp.float32),
            pltpu.VMEM((N,), jnp.float32),
            pltpu.VMEM((256,), jnp.float32),
            pltpu.VMEM((256,), jnp.float32),
            pltpu.VMEM((CAP,), jnp.float32),
            pltpu.VMEM((CAP,), jnp.int32),
            pltpu.VMEM((OUT_CH, 80), jnp.int32),
            pltpu.VMEM((OUT_CH * 32,), jnp.int32),
            pltpu.VMEM((OUT_CH * 48,), jnp.int32),
            pltpu.SemaphoreType.DMA,
            pltpu.SemaphoreType.DMA,
            pltpu.SemaphoreType.DMA,
            pltpu.SemaphoreType.DMA,
        ],
        compiler_params=pltpu.CompilerParams(needs_layout_passes=False),
    )
    def topk(pd_hbm, cm_hbm, out_hbm, out1_hbm, out2_hbm, row0, row1,
             cmb0, cmb1, cand_v, cand_i, out_v, out_v1, out_v2,
             semr0, semr1, semc0, semc1):
        wid = lax.axis_index("s") * NC + lax.axis_index("c")
        base = wid * ROWS_PER_W
        joff = (base // N) * N

        iota = lax.broadcasted_iota(jnp.int32, (16,), 0)

        def issue(p, rowb, cmb, semr, semc):
            pc = jnp.minimum(p, ROWS_PER_W - 1)
            pltpu.async_copy(pd_hbm.at[base + pc, :], rowb, semr)
            pltpu.async_copy(cm_hbm.at[base + pc, :], cmb, semc)

        def waitb(rowb, cmb, semr, semc):
            pltpu.make_async_copy(pd_hbm.at[base, :], rowb, semr).wait()
            pltpu.make_async_copy(cm_hbm.at[base, :], cmb, semc).wait()

        def process(p, rowb, cmb):
            neg = jnp.full((16,), _NEG, jnp.float32)
            r = [neg, neg, neg, neg, neg]
            for i in range(16):
                v = cmb[pl.ds(i * 16, 16)]
                for s in range(5):
                    hi = jnp.maximum(r[s], v)
                    v = jnp.minimum(r[s], v)
                    r[s] = hi
            tv = _lane_min_splat(r[4])

            def compact(jb, off):
                for u in range(8):
                    j0 = (jb * 8 + u) * 16
                    v = rowb[pl.ds(j0, 16)]
                    m = v >= tv
                    plsc.store_compressed(cand_v.at[pl.ds(off, 16)], v, mask=m)
                    plsc.store_compressed(cand_i.at[pl.ds(off, 16)],
                                          iota + j0, mask=m)
                    off = off + plsc.all_reduce_population_count(m)[0]
                return off

            off = lax.fori_loop(0, 16, compact, jnp.int32(0), unroll=False)
            cand_v[pl.ds(off, 16)] = neg
            cand_v[pl.ds(off + 16, 16)] = neg

            zero = jnp.zeros((16,), jnp.int32)
            init = (neg, neg, neg, neg, neg, zero, zero, zero, zero, zero)

            def sel(i2, carry):
                Lk = list(carry[0:5])
                Lp = list(carry[5:10])
                v1 = cand_v[pl.ds(i2 * 32, 16)]
                p1 = cand_i[pl.ds(i2 * 32, 16)]
                v2 = cand_v[pl.ds(i2 * 32 + 16, 16)]
                p2 = cand_i[pl.ds(i2 * 32 + 16, 16)]
                lminv = Lk[4].at[jnp.full((16,), 15, jnp.int32)].get(
                    mode="promise_in_bounds")

                def do_merge(_):
                    s1k, s1p = plsc.sort_key_val(v1, p1, descending=False)
                    s2k, s2p = plsc.sort_key_val(v2, p2, descending=False)
                    r2k = lax.rev(s2k, (0,))
                    r2p = lax.rev(s2p, (0,))
                    m = s1k >= r2k
                    hk = jnp.where(m, s1k, r2k)
                    hp = jnp.where(m, s1p, r2p)
                    lk = jnp.where(m, r2k, s1k)
                    lp = jnp.where(m, r2p, s1p)
                    wlk, wlp = plsc.sort_key_val(lk, lp, descending=False)
                    whk, whp = plsc.sort_key_val(hk, hp, descending=False)
                    nk, np_ = _merge_pair(Lk, Lp, wlk, wlp, whk, whp)
                    return tuple(nk) + tuple(np_)

                def skip(_):
                    return tuple(Lk) + tuple(Lp)

                hit = jnp.any(jnp.maximum(v1, v2) >= lminv)
                return lax.cond(hit, do_merge, skip, 0)

            nv2 = (off + 31) // 32
            fin = lax.fori_loop(0, nv2, sel, init, unroll=False)

            prow = p % OUT_CH
            gl = [fin[5 + g] + joff for g in range(5)]
            for g in range(5):
                out_v[prow, pl.ds(g * 16, 16)] = gl[g]
            selfv = jnp.full((16,), 0, jnp.int32) + (base + p)
            out_v1[pl.ds(prow * 32, 16)] = gl[0]
            out_v1[pl.ds(prow * 32 + 16, 16)] = jnp.where(iota < 4, gl[1],
                                                          selfv)
            out_v2[pl.ds(prow * 48, 16)] = gl[0]
            out_v2[pl.ds(prow * 48 + 16, 16)] = gl[1]
            out_v2[pl.ds(prow * 48 + 32, 16)] = jnp.where(iota < 8, gl[2],
                                                          selfv)

        issue(0, row0, cmb0, semr0, semc0)
        issue(1, row1, cmb1, semr1, semc1)

        def flush_chunk(o, _):
            def pair(tt, _):
                p0 = o * OUT_CH + 2 * tt
                waitb(row0, cmb0, semr0, semc0)
                process(p0, row0, cmb0)
                issue(p0 + 2, row0, cmb0, semr0, semc0)
                waitb(row1, cmb1, semr1, semc1)
                process(p0 + 1, row1, cmb1)
                issue(p0 + 3, row1, cmb1, semr1, semc1)
                return 0

            lax.fori_loop(0, OUT_CH // 2, pair, 0, unroll=False)
            pltpu.sync_copy(
                out_v, out_hbm.at[pl.ds(base + o * OUT_CH, OUT_CH), :])
            pltpu.sync_copy(
                out_v1,
                out1_hbm.at[pl.ds((base + o * OUT_CH) * 32, OUT_CH * 32)])
            pltpu.sync_copy(
                out_v2,
                out2_hbm.at[pl.ds((base + o * OUT_CH) * 48, OUT_CH * 48)])
            return 0

        lax.fori_loop(0, ROWS_PER_W // OUT_CH, flush_chunk, 0, unroll=False)
        waitb(row0, cmb0, semr0, semc0)
        waitb(row1, cmb1, semr1, semc1)

    return topk


def _sc_topk(pd, cm):
    return _make_topk()(pd.reshape(NROWS, N), cm.reshape(NROWS, 256))


def kernel(x, W1a, g1a, b1a, W1b, g1b, b1b, W2a, g2a, b2a, W2b, g2b, b2b,
           W3a, g3a, b3a, W3b, g3b, b3b, W4a, g4a, b4a, W4b, g4b, b4b,
           W5a, g5a, b5a, W5b, g5b, b5b):
    xt = jnp.transpose(x, (0, 2, 1))
    x8 = jnp.pad(x, ((0, 0), (0, 0), (0, 5)))
    xt8 = jnp.pad(xt, ((0, 0), (0, 5), (0, 0)))
    norms = jnp.sum(x * x, axis=-1)[:, None, :]
    Wa8 = jnp.pad(W1a, ((0, 0), (0, 5)))
    Wb8 = jnp.pad(W1b, ((0, 0), (0, 5)))

    pd, cm = _pd_chunkmax(x8, xt8, norms)
    idxg, idx1f, idx2f = _sc_topk(pd, cm)

    fa1T, fb1 = _conv1(xt8, Wa8, g1a, b1a, Wb8, g1b, b1b, 128)
    agg1 = _sc_agg_batch(fa1T, idx1f, 32, 4, 64).reshape(B, N, 64)
    fa2T, fb2, x1 = _conv_stage(agg1, fb1, W2a, g2a, b2a, W2b, g2b, b2b, 8, 128)
    agg2 = _sc_agg_batch(fa2T, idx2f, 48, 2, 64).reshape(B, N, 64)
    fa3T, fb3, x2 = _conv_stage(agg2, fb2, W3a, g3a, b3a, W3b, g3b, b3b, 8, 128)
    agg3 = _sc_agg(fa3T, idxg, K + 2 * P, 128).reshape(B, N, 128)
    fa4P, fb4, x3 = _conv_stage4(agg3, fb3, W4a, g4a, b4a, W4b, g4b, b4b, 16)
    agg4 = _sc_agg_packed(fa4P, idxg, K + 3 * P).reshape(B, N, 128)

    return _head(agg4, fb4, x1, x2, x3, W5a, g5a, b5a, W5b, g5b, b5b)
